# baseline (speedup 1.0000x reference)
import jax
import jax.numpy as jnp
from jax.experimental import pallas as pl
from jax.experimental.pallas import tpu as pltpu

_FENCE_T = 36
_GR = 16
_CB = 2048
_K = 4


def _lstm_lm_kernel(S, E, Hp, G, V):
    gr = min(_GR, S)
    cb = min(_CB, V)
    n_chunks = V // cb

    def body(sent_ref, table_hbm, wi_ref, wh_ref, b_ref, fcw_ref, fcb_ref,
             out_hbm, emb_ref, out_scr, hid_ref, emb_sem, out_sem):

        copies = []
        for t in range(S):
            base = pl.multiple_of((sent_ref[t] >> 3) << 3, 8)
            cp = pltpu.make_async_copy(
                table_hbm.at[pl.ds(base, 8), :], emb_ref.at[t], emb_sem)
            cp.start()
            copies.append(cp)
        for cp in copies:
            cp.wait()

        iota8 = jax.lax.broadcasted_iota(jnp.int32, (8, E), 0)
        wh = wh_ref[...]
        bias = b_ref[...]

        h = jnp.zeros((1, Hp), jnp.float32)
        c = jnp.zeros((1, Hp), jnp.float32)
        out_copies = []

        state = {}

        def group_work(g):
            r0 = gr * g
            hg = hid_ref[r0:r0 + gr, :]
            nlt = cb // 128
            for j in range(n_chunks):
                c0 = cb * j
                a = jnp.dot(hg, fcw_ref[:, c0:c0 + cb],
                            preferred_element_type=jnp.float32) \
                    + fcb_ref[:, c0:c0 + cb]
                a = jnp.maximum(a, 0.0)
                out_scr[r0:r0 + gr, c0:c0 + cb] = a
                f = a[:, 0:128]
                for k in range(1, nlt):
                    f = jnp.maximum(f, a[:, 128 * k:128 * (k + 1)])
                state[g] = f if j == 0 else jnp.maximum(state[g], f)
                yield None
            m = jnp.max(state[g], axis=1, keepdims=True)
            for j in range(n_chunks):
                c0 = cb * j
                e = jnp.exp(out_scr[r0:r0 + gr, c0:c0 + cb] - m)
                sf = e[:, 0:128]
                for k in range(1, nlt):
                    sf = sf + e[:, 128 * k:128 * (k + 1)]
                state[g, "s"] = sf if j == 0 else state[g, "s"] + sf
                yield None
            lse = jnp.log(jnp.sum(state[g, "s"], axis=1, keepdims=True)) + m
            for j in range(n_chunks):
                c0 = cb * j
                out_scr[r0:r0 + gr, c0:c0 + cb] = \
                    out_scr[r0:r0 + gr, c0:c0 + cb] - lse
                yield None
            cp = pltpu.make_async_copy(
                out_scr.at[pl.ds(r0, gr), :],
                out_hbm.at[pl.ds(r0, gr), :], out_sem)
            cp.start()
            out_copies.append(cp)
            yield None

        queue = []

        for t in range(S):
            mask = (iota8 == (sent_ref[t] & 7)).astype(jnp.float32)
            emb_t = jnp.sum(emb_ref[t] * mask, axis=0, keepdims=True)
            gin = jnp.dot(emb_t, wi_ref[...],
                          preferred_element_type=jnp.float32) + bias

            gates = gin + jnp.dot(h, wh, preferred_element_type=jnp.float32)
            sg = jax.nn.sigmoid(gates)
            i_g = sg[:, 0 * Hp:1 * Hp]
            f_g = sg[:, 1 * Hp:2 * Hp]
            g_g = 2.0 * sg[:, 2 * Hp:3 * Hp] - 1.0
            o_g = sg[:, 3 * Hp:4 * Hp]
            c = f_g * c + i_g * g_g
            h = o_g * jnp.tanh(c)
            hid_ref[t:t + 1, :] = h

            if (t + 1) % gr == 0:
                queue.append(group_work((t + 1) // gr - 1))
            if t > min(_FENCE_T, S - 1):
                emitted = 0
                while queue and emitted < _K:
                    try:
                        next(queue[0])
                        emitted += 1
                    except StopIteration:
                        queue.pop(0)

        while queue:
            try:
                next(queue[0])
            except StopIteration:
                queue.pop(0)

        for cp in out_copies:
            cp.wait()

    return body


def kernel(sentence, table, Wi, Wh, b, fcw, fcb):
    sent = sentence.reshape(-1).astype(jnp.int32)
    S = sent.shape[0]
    V, E = table.shape
    Hp, G = Wh.shape
    Vout = fcw.shape[1]

    def full(shape):
        return pl.BlockSpec(shape, lambda i, s: (0,) * len(shape))

    grid_spec = pltpu.PrefetchScalarGridSpec(
        num_scalar_prefetch=1,
        grid=(1,),
        in_specs=[
            pl.BlockSpec(memory_space=pl.ANY),
            full((E, G)),
            full((Hp, G)),
            full((1, G)),
            full((Hp, Vout)),
            full((1, Vout)),
        ],
        out_specs=pl.BlockSpec(memory_space=pl.ANY),
        scratch_shapes=[
            pltpu.VMEM((S, 8, E), jnp.float32),
            pltpu.VMEM((S, Vout), jnp.float32),
            pltpu.VMEM((S, Hp), jnp.float32),
            pltpu.SemaphoreType.DMA,
            pltpu.SemaphoreType.DMA,
        ],
    )
    return pl.pallas_call(
        _lstm_lm_kernel(S, E, Hp, G, Vout),
        out_shape=jax.ShapeDtypeStruct((S, Vout), jnp.float32),
        grid_spec=grid_spec,
        compiler_params=pltpu.CompilerParams(
            dimension_semantics=("arbitrary",),
            vmem_limit_bytes=100 * 1024 * 1024,
        ),
    )(sent, table, Wi, Wh, b, fcw, fcb)


# speedup vs baseline: 1.0056x; 1.0056x over previous
import jax
import jax.numpy as jnp
from jax.experimental import pallas as pl
from jax.experimental.pallas import tpu as pltpu

_FENCE_T = 36
_GR = 16
_CB = 2048
_K = 4


def _lstm_lm_kernel(S, E, Hp, G, V):
    gr = min(_GR, S)
    cb = min(_CB, V)
    n_chunks = V // cb

    def body(sent_ref, table_hbm, wi_ref, wh_ref, b_ref, fcw_hbm, fcb_ref,
             out_hbm, emb_ref, fcw_ref, out_scr, hid_ref, emb_sem, fcw_sem, out_sem):
        fcw_copies = []
        nsplit = 8
        rows = Hp // nsplit
        for k in range(nsplit):
            cpw = pltpu.make_async_copy(
                fcw_hbm.at[pl.ds(rows * k, rows), :],
                fcw_ref.at[pl.ds(rows * k, rows), :], fcw_sem)
            cpw.start()
            fcw_copies.append(cpw)

        copies = []
        for t in range(S):
            base = pl.multiple_of((sent_ref[t] >> 3) << 3, 8)
            cp = pltpu.make_async_copy(
                table_hbm.at[pl.ds(base, 8), :], emb_ref.at[t], emb_sem)
            cp.start()
            copies.append(cp)
        for cp in copies:
            cp.wait()

        iota8 = jax.lax.broadcasted_iota(jnp.int32, (8, E), 0)
        wh = wh_ref[...]
        bias = b_ref[...]

        h = jnp.zeros((1, Hp), jnp.float32)
        c = jnp.zeros((1, Hp), jnp.float32)
        out_copies = []

        state = {}

        def group_work(g):
            r0 = gr * g
            hg = hid_ref[r0:r0 + gr, :]
            nlt = cb // 128
            for j in range(n_chunks):
                c0 = cb * j
                a = jnp.dot(hg, fcw_ref[:, c0:c0 + cb],
                            preferred_element_type=jnp.float32) \
                    + fcb_ref[:, c0:c0 + cb]
                a = jnp.maximum(a, 0.0)
                out_scr[r0:r0 + gr, c0:c0 + cb] = a
                f = a[:, 0:128]
                for k in range(1, nlt):
                    f = jnp.maximum(f, a[:, 128 * k:128 * (k + 1)])
                state[g] = f if j == 0 else jnp.maximum(state[g], f)
                yield None
            m = jnp.max(state[g], axis=1, keepdims=True)
            for j in range(n_chunks):
                c0 = cb * j
                e = jnp.exp(out_scr[r0:r0 + gr, c0:c0 + cb] - m)
                sf = e[:, 0:128]
                for k in range(1, nlt):
                    sf = sf + e[:, 128 * k:128 * (k + 1)]
                state[g, "s"] = sf if j == 0 else state[g, "s"] + sf
                yield None
            lse = jnp.log(jnp.sum(state[g, "s"], axis=1, keepdims=True)) + m
            for j in range(n_chunks):
                c0 = cb * j
                out_scr[r0:r0 + gr, c0:c0 + cb] = \
                    out_scr[r0:r0 + gr, c0:c0 + cb] - lse
                yield None
            cp = pltpu.make_async_copy(
                out_scr.at[pl.ds(r0, gr), :],
                out_hbm.at[pl.ds(r0, gr), :], out_sem)
            cp.start()
            out_copies.append(cp)
            yield None

        queue = []

        for t in range(S):
            mask = (iota8 == (sent_ref[t] & 7)).astype(jnp.float32)
            emb_t = jnp.sum(emb_ref[t] * mask, axis=0, keepdims=True)
            gin = jnp.dot(emb_t, wi_ref[...],
                          preferred_element_type=jnp.float32) + bias

            gates = gin + jnp.dot(h, wh, preferred_element_type=jnp.float32)
            sg = jax.nn.sigmoid(gates)
            i_g = sg[:, 0 * Hp:1 * Hp]
            f_g = sg[:, 1 * Hp:2 * Hp]
            g_g = 2.0 * sg[:, 2 * Hp:3 * Hp] - 1.0
            o_g = sg[:, 3 * Hp:4 * Hp]
            c = f_g * c + i_g * g_g
            h = o_g * jnp.tanh(c)
            hid_ref[t:t + 1, :] = h

            if (t + 1) % gr == 0:
                queue.append(group_work((t + 1) // gr - 1))
            if t == min(_FENCE_T, S - 1):
                for cpw in fcw_copies:
                    cpw.wait()
            if t > min(_FENCE_T, S - 1):
                emitted = 0
                while queue and emitted < _K:
                    try:
                        next(queue[0])
                        emitted += 1
                    except StopIteration:
                        queue.pop(0)

        while queue:
            try:
                next(queue[0])
            except StopIteration:
                queue.pop(0)

        for cp in out_copies:
            cp.wait()

    return body


def kernel(sentence, table, Wi, Wh, b, fcw, fcb):
    sent = sentence.reshape(-1).astype(jnp.int32)
    S = sent.shape[0]
    V, E = table.shape
    Hp, G = Wh.shape
    Vout = fcw.shape[1]

    def full(shape):
        return pl.BlockSpec(shape, lambda i, s: (0,) * len(shape))

    grid_spec = pltpu.PrefetchScalarGridSpec(
        num_scalar_prefetch=1,
        grid=(1,),
        in_specs=[
            pl.BlockSpec(memory_space=pl.ANY),
            full((E, G)),
            full((Hp, G)),
            full((1, G)),
            pl.BlockSpec(memory_space=pl.ANY),
            full((1, Vout)),
        ],
        out_specs=pl.BlockSpec(memory_space=pl.ANY),
        scratch_shapes=[
            pltpu.VMEM((S, 8, E), jnp.float32),
            pltpu.VMEM((Hp, Vout), jnp.float32),
            pltpu.VMEM((S, Vout), jnp.float32),
            pltpu.VMEM((S, Hp), jnp.float32),
            pltpu.SemaphoreType.DMA,
            pltpu.SemaphoreType.DMA,
            pltpu.SemaphoreType.DMA,
        ],
    )
    return pl.pallas_call(
        _lstm_lm_kernel(S, E, Hp, G, Vout),
        out_shape=jax.ShapeDtypeStruct((S, Vout), jnp.float32),
        grid_spec=grid_spec,
        compiler_params=pltpu.CompilerParams(
            dimension_semantics=("arbitrary",),
            vmem_limit_bytes=100 * 1024 * 1024,
        ),
    )(sent, table, Wi, Wh, b, fcw, fcb)
